# baseline (speedup 1.0000x reference)
import functools

import jax
import jax.numpy as jnp
from jax import lax
from jax.experimental import pallas as pl
from jax.experimental.pallas import tpu as pltpu
from jax.experimental.pallas import tpu_sc as plsc

_N = 10000
_NP = 10240
_BM = 256
_MG = _NP // _BM
_NC = 2
_NS = 16
_NW = _NC * _NS
_STRIPE = _NP // _NS
_K = 128
_W = 64

_f32 = jnp.float32

_SC_PARAMS = pltpu.CompilerParams(use_tc_tiling_on_sc=False)


def _deg_body(dstF, out, dstv, onesv, zbuf, acc, *, ch):
    cid = lax.axis_index("c")
    sid = lax.axis_index("s")
    hc = ch // 2
    pltpu.sync_copy(dstF.at[sid, pl.ds(cid * hc, hc)], dstv)

    def _fill(r, _):
        onesv[r, pl.ds(0, 16)] = jnp.full((16,), 1.0, _f32)
        zbuf[r, pl.ds(0, 16)] = jnp.zeros((16,), _f32)
        return 0

    lax.fori_loop(0, _K, _fill, 0)
    for z in range(_STRIPE // _K):
        pltpu.sync_copy(zbuf, acc.at[pl.ds(sid * _STRIPE + z * _K, _K)])
    plsc.subcore_barrier()

    def _chunk(j, _):
        pltpu.sync_copy(onesv, acc.at[dstv.at[j]], add=True)
        return 0

    lax.fori_loop(0, hc, _chunk, 0)
    plsc.subcore_barrier()
    pltpu.sync_copy(acc.at[pl.ds(sid * _STRIPE, _STRIPE)],
                    out.at[cid, pl.ds(sid * _STRIPE, _STRIPE)])


def _sc_degree(dstF, ch):
    mesh = plsc.VectorSubcoreMesh(core_axis_name="c", subcore_axis_name="s")
    kern = pl.kernel(
        functools.partial(_deg_body, ch=ch),
        out_type=jax.ShapeDtypeStruct((_NC, _NP, 16), _f32),
        mesh=mesh,
        scratch_types=[
            pltpu.VMEM((ch // 2, _K), jnp.int32),
            pltpu.VMEM((_K, 16), _f32),
            pltpu.VMEM((_K, 16), _f32),
            pltpu.VMEM_SHARED((_NP, 16), _f32),
        ],
        compiler_params=_SC_PARAMS,
    )
    return kern(dstF)


def _prop_body(table, srcF, dstF, out, dstv, srcsh,
               b0, b1, b2, b3,
               acc, gsA, gsB, ssA, ssB, *, w, ch, slabs0, slabs1):
    cid = lax.axis_index("c")
    sid = lax.axis_index("s")
    bufs = (b0, b1, b2, b3)
    ng = ch // 4
    pltpu.sync_copy(srcF.at[sid], srcsh)
    pltpu.sync_copy(dstF.at[sid], dstv)

    def _fill(r, _):
        for c in range(w // 16):
            b0[r, pl.ds(c * 16, 16)] = jnp.zeros((16,), _f32)
        return 0

    def _gather(j, b, sem):
        pltpu.async_copy(table.at[srcsh.at[j]], bufs[b], sem)

    def _drain(sem, b):
        pltpu.make_async_copy(table.at[pl.ds(0, _K)], bufs[b], sem).wait()

    def _scatter(j, b, sem):
        del sem
        pltpu.sync_copy(bufs[b], acc.at[dstv.at[j]], add=True)

    def _group(g, _):
        j0 = 4 * g

        for b in range(2):
            _gather(j0 + 2 + b, 2 + b, gsB)
        for b in range(2):
            _drain(gsA, b)
        for b in range(2):
            _scatter(j0 + b, b, ssA)

        @pl.when(g + 1 < ng)
        def _():
            for b in range(2):
                _gather(j0 + 4 + b, b, gsA)
        for b in range(2):
            _drain(gsB, 2 + b)
        for b in range(2):
            _scatter(j0 + 2 + b, 2 + b, ssB)
        return 0

    def _one_slab(s, off):
        def _shift(j, _):
            for c in range(_K // 16):
                srcsh[j, pl.ds(c * 16, 16)] = \
                    srcsh[j, pl.ds(c * 16, 16)] + off
            return 0

        if off is not None:
            lax.fori_loop(0, ch, _shift, 0)
        lax.fori_loop(0, _K, _fill, 0)
        for z in range(_STRIPE // _K):
            pltpu.sync_copy(b0, acc.at[pl.ds(sid * _STRIPE + z * _K, _K)])
        for b in range(2):
            _gather(b, b, gsA)
        plsc.subcore_barrier()
        lax.fori_loop(0, ng, _group, 0)
        plsc.subcore_barrier()
        pltpu.sync_copy(acc.at[pl.ds(sid * _STRIPE, _STRIPE)],
                        out.at[s, pl.ds(sid * _STRIPE, _STRIPE)])

    num_slabs = len(slabs0) + len(slabs1)
    if slabs1 and len(slabs0) == len(slabs1):
        half = num_slabs // 2
        base = cid * half
        for i in range(half):
            s = base + i
            off = base * jnp.int32(_NP) if i == 0 else jnp.int32(_NP)
            _one_slab(s, off)
    else:
        for cv, my_slabs in ((0, slabs0), (1, slabs1)):
            if not my_slabs:
                continue

            @pl.when(cid == cv)
            def _(my_slabs=my_slabs):
                prev = 0
                for s in my_slabs:
                    d = s - prev
                    _one_slab(s, jnp.int32(d * _NP) if d else None)
                    prev = s


def _sc_prop(table_flat, srcF, dstF, num_slabs, w, ch, n1):
    slabs0 = tuple(range(num_slabs - n1))
    slabs1 = tuple(range(num_slabs - n1, num_slabs))
    mesh = plsc.VectorSubcoreMesh(core_axis_name="c", subcore_axis_name="s")
    kern = pl.kernel(
        functools.partial(_prop_body, w=w, ch=ch,
                          slabs0=slabs0, slabs1=slabs1),
        out_type=jax.ShapeDtypeStruct((num_slabs, _NP, w), _f32),
        mesh=mesh,
        scratch_types=[
            pltpu.VMEM((ch, _K), jnp.int32),
            pltpu.VMEM((ch, _K), jnp.int32),
        ] + [pltpu.VMEM((_K, w), _f32) for _ in range(4)] + [
            pltpu.VMEM_SHARED((_NP, w), _f32),
            pltpu.SemaphoreType.DMA,
            pltpu.SemaphoreType.DMA,
            pltpu.SemaphoreType.DMA,
            pltpu.SemaphoreType.DMA,
        ],
        compiler_params=_SC_PARAMS,
    )
    return kern(table_flat, srcF, dstF)


_TC_PARAMS = pltpu.CompilerParams(dimension_semantics=("arbitrary",))


def _u_body(degp, xpad, u_ref, xp_ref):
    deg = degp[0, :, 0:1] + degp[1, :, 0:1] + 1.0
    u = lax.rsqrt(deg)
    u_ref[...] = u
    xs = xpad[...] * u
    xp_ref[0] = xs[:, :_W]
    xp_ref[1] = xs[:, _W:]


def _tc_u(degp, xpad):
    return pl.pallas_call(
        _u_body,
        grid=(_MG,),
        in_specs=[
            pl.BlockSpec((_NC, _BM, 16), lambda i: (0, i, 0)),
            pl.BlockSpec((_BM, 128), lambda i: (i, 0)),
        ],
        out_specs=[
            pl.BlockSpec((_BM, 1), lambda i: (i, 0)),
            pl.BlockSpec((2, _BM, _W), lambda i: (0, i, 0)),
        ],
        out_shape=[
            jax.ShapeDtypeStruct((_NP, 1), _f32),
            jax.ShapeDtypeStruct((2, _NP, _W), _f32),
        ],
        compiler_params=_TC_PARAMS,
    )(degp, xpad)


def _l1_body(p1, xp, u, w1, b1, out):
    z = jnp.concatenate(
        [p1[k] + xp[k] for k in range(2)], axis=1) * u[...]
    h = jnp.dot(z, w1[...], preferred_element_type=_f32) + b1[...]
    out[...] = jnp.maximum(h, 0.0)


def _tc_l1(p1, xp, u, w1, b1):
    dout = w1.shape[1]
    return pl.pallas_call(
        _l1_body,
        grid=(_MG,),
        in_specs=[
            pl.BlockSpec((2, _BM, _W), lambda i: (0, i, 0)),
            pl.BlockSpec((2, _BM, _W), lambda i: (0, i, 0)),
            pl.BlockSpec((_BM, 1), lambda i: (i, 0)),
            pl.BlockSpec(w1.shape, lambda i: (0, 0)),
            pl.BlockSpec(b1.shape, lambda i: (0, 0)),
        ],
        out_specs=pl.BlockSpec((_BM, dout), lambda i: (i, 0)),
        out_shape=jax.ShapeDtypeStruct((_NP, dout), _f32),
        compiler_params=_TC_PARAMS,
    )(p1, xp, u, w1, b1)


def _l2_body(h1, u, w2, out, *, ns):
    g = jnp.dot(h1[...], w2[...], preferred_element_type=_f32) * u[...]
    for j in range(ns):
        out[j] = g[:, _W * j:_W * (j + 1)]


def _tc_l2(h1, u, w2):
    din = w2.shape[0]
    ns = w2.shape[1] // _W
    return pl.pallas_call(
        functools.partial(_l2_body, ns=ns),
        grid=(_MG,),
        in_specs=[
            pl.BlockSpec((_BM, din), lambda i: (i, 0)),
            pl.BlockSpec((_BM, 1), lambda i: (i, 0)),
            pl.BlockSpec(w2.shape, lambda i: (0, 0)),
        ],
        out_specs=pl.BlockSpec((ns, _BM, _W), lambda i: (0, i, 0)),
        out_shape=jax.ShapeDtypeStruct((ns, _NP, _W), _f32),
        compiler_params=_TC_PARAMS,
    )(h1, u, w2)


def _mid_body(pp, gg, u, b, w, out, *, ks, dout):
    uu = u[...]
    acc = None
    wm = w[...]
    for k in range(0, ks, 2):
        hk = jnp.concatenate(
            [jnp.maximum((pp[k + j] + gg[k + j]) * uu
                         + b[k + j], 0.0) for j in range(2)], axis=1)
        t = jnp.dot(hk, wm[_W * k:_W * (k + 2), :],
                    preferred_element_type=_f32)
        acc = t if acc is None else acc + t
    g = acc * uu
    if dout > _W:
        for j in range(dout // _W):
            out[j] = g[:, _W * j:_W * (j + 1)]
    else:
        out[...] = g


def _tc_mid(pp, gg, u, b, w):
    ks = gg.shape[0]
    dout = w.shape[1]
    if dout > _W:
        out_spec = pl.BlockSpec((dout // _W, _BM, _W), lambda i: (0, i, 0))
        out_shape = jax.ShapeDtypeStruct((dout // _W, _NP, _W), _f32)
    else:
        out_spec = pl.BlockSpec((_BM, dout), lambda i: (i, 0))
        out_shape = jax.ShapeDtypeStruct((_NP, dout), _f32)
    return pl.pallas_call(
        functools.partial(_mid_body, ks=ks, dout=dout),
        grid=(_MG,),
        in_specs=[
            pl.BlockSpec((ks, _BM, _W), lambda i: (0, i, 0)),
            pl.BlockSpec((ks, _BM, _W), lambda i: (0, i, 0)),
            pl.BlockSpec((_BM, 1), lambda i: (i, 0)),
            pl.BlockSpec(b.shape, lambda i: (0, 0)),
            pl.BlockSpec(w.shape, lambda i: (0, 0)),
        ],
        out_specs=out_spec,
        out_shape=out_shape,
        compiler_params=_TC_PARAMS,
    )(pp, gg, u, b, w)


def _nar_body(pp, gg, u, b, w, out):
    uu = u[...]
    hk = jnp.maximum((pp[...] + gg[...]) * uu + b[...], 0.0)
    out[...] = jnp.dot(hk, w[...], preferred_element_type=_f32) * uu


def _tc_narrow(pp, gg, u, b, w):
    din = gg.shape[1]
    dout = w.shape[1]
    return pl.pallas_call(
        _nar_body,
        grid=(_MG,),
        in_specs=[
            pl.BlockSpec((_BM, din), lambda i: (i, 0)),
            pl.BlockSpec((_BM, din), lambda i: (i, 0)),
            pl.BlockSpec((_BM, 1), lambda i: (i, 0)),
            pl.BlockSpec(b.shape, lambda i: (0, 0)),
            pl.BlockSpec(w.shape, lambda i: (0, 0)),
        ],
        out_specs=pl.BlockSpec((_BM, dout), lambda i: (i, 0)),
        out_shape=jax.ShapeDtypeStruct((_NP, dout), _f32),
        compiler_params=_TC_PARAMS,
    )(pp, gg, u, b, w)


def _fin_body(pp, gg, u, b5, wl, bl, out, acc):
    m = pl.program_id(0)
    h5 = (pp[...] + gg[...]) * u[...] + b5[...]
    rows = m * _BM + lax.broadcasted_iota(jnp.int32, (_BM, 1), 0)
    h5 = jnp.where(rows < _N, h5, 0.0)
    part = jnp.sum(h5, axis=0, keepdims=True)

    @pl.when(m == 0)
    def _():
        acc[...] = jnp.zeros_like(acc)

    acc[...] += part

    @pl.when(m == _MG - 1)
    def _():
        pooled = acc[...] * (1.0 / _N)
        out[...] = jnp.dot(pooled, wl[...], preferred_element_type=_f32) \
            + bl[...]


def _tc_final(pp, gg, u, b5, wl, bl):
    din = gg.shape[1]
    return pl.pallas_call(
        _fin_body,
        grid=(_MG,),
        in_specs=[
            pl.BlockSpec((_BM, din), lambda i: (i, 0)),
            pl.BlockSpec((_BM, din), lambda i: (i, 0)),
            pl.BlockSpec((_BM, 1), lambda i: (i, 0)),
            pl.BlockSpec(b5.shape, lambda i: (0, 0)),
            pl.BlockSpec(wl.shape, lambda i: (0, 0)),
            pl.BlockSpec(bl.shape, lambda i: (0, 0)),
        ],
        out_specs=pl.BlockSpec((1, 3), lambda i: (0, 0)),
        out_shape=jax.ShapeDtypeStruct((1, 3), _f32),
        scratch_shapes=[pltpu.VMEM((1, din), _f32)],
        compiler_params=_TC_PARAMS,
    )(pp, gg, u, b5, wl, bl)


def kernel(x, edge_index, W1, b1, W2, b2, W3, b3, W4, b4, W5, b5, Wl, bl):
    e = edge_index.shape[1]
    ch = -(-e // (_K * _NS))
    ch = -(-ch // 4) * 4
    tot = _NS * ch
    pad = tot * _K - e

    src = jnp.concatenate([edge_index[0], jnp.zeros((pad,), jnp.int32)])
    dst = jnp.concatenate([edge_index[1], jnp.full((pad,), _N, jnp.int32)])
    srcF = src.reshape(_NS, ch, _K)
    dstF = dst.reshape(_NS, ch, _K)

    xpad = jnp.pad(x, ((0, _NP - _N), (0, 0)))

    degp = _sc_degree(dstF, ch)
    u, xp = _tc_u(degp, xpad)

    p1 = _sc_prop(xp.reshape(-1, _W), srcF, dstF, 2, _W, ch, 1)
    h1 = _tc_l1(p1, xp, u, W1, b1.reshape(1, -1))

    g2 = _tc_l2(h1, u, W2)
    p2 = _sc_prop(g2.reshape(-1, _W), srcF, dstF, 16, _W, ch, 8)

    g3 = _tc_mid(p2, g2, u, b2.reshape(-1, _W), W3)
    p3 = _sc_prop(g3.reshape(-1, _W), srcF, dstF, 4, _W, ch, 2)

    g4 = _tc_mid(p3, g3, u, b3.reshape(-1, _W), W4)
    p4 = _sc_prop(g4, srcF, dstF, 1, _W, ch, 0)[0]

    g5 = _tc_narrow(p4, g4, u, b4.reshape(1, -1), W5)
    p5 = _sc_prop(g5, srcF, dstF, 1, 32, ch, 0)[0]

    return _tc_final(p5, g5, u, b5.reshape(1, -1), Wl, bl.reshape(1, -1))


# speedup vs baseline: 1.0236x; 1.0236x over previous
import functools

import jax
import jax.numpy as jnp
from jax import lax
from jax.experimental import pallas as pl
from jax.experimental.pallas import tpu as pltpu
from jax.experimental.pallas import tpu_sc as plsc

_N = 10000
_NP = 10240
_BM = 256
_MG = _NP // _BM
_NC = 2
_NS = 16
_NW = _NC * _NS
_STRIPE = _NP // _NS
_K = 128
_W = 64

_f32 = jnp.float32

_SC_PARAMS = pltpu.CompilerParams(use_tc_tiling_on_sc=False)


def _deg_body(dstF, out, dstv, onesv, zbuf, acc, *, ch):
    cid = lax.axis_index("c")
    sid = lax.axis_index("s")
    hc = ch // 2
    pltpu.sync_copy(dstF.at[sid, pl.ds(cid * hc, hc)], dstv)

    def _fill(r, _):
        onesv[r, pl.ds(0, 16)] = jnp.full((16,), 1.0, _f32)
        zbuf[r, pl.ds(0, 16)] = jnp.zeros((16,), _f32)
        return 0

    lax.fori_loop(0, _K, _fill, 0)
    for z in range(_STRIPE // _K):
        pltpu.sync_copy(zbuf, acc.at[pl.ds(sid * _STRIPE + z * _K, _K)])
    plsc.subcore_barrier()

    def _chunk(j, _):
        pltpu.sync_copy(onesv, acc.at[dstv.at[j]], add=True)
        return 0

    lax.fori_loop(0, hc, _chunk, 0)
    plsc.subcore_barrier()
    pltpu.sync_copy(acc.at[pl.ds(sid * _STRIPE, _STRIPE)],
                    out.at[cid, pl.ds(sid * _STRIPE, _STRIPE)])


def _sc_degree(dstF, ch):
    mesh = plsc.VectorSubcoreMesh(core_axis_name="c", subcore_axis_name="s")
    kern = pl.kernel(
        functools.partial(_deg_body, ch=ch),
        out_type=jax.ShapeDtypeStruct((_NC, _NP, 16), _f32),
        mesh=mesh,
        scratch_types=[
            pltpu.VMEM((ch // 2, _K), jnp.int32),
            pltpu.VMEM((_K, 16), _f32),
            pltpu.VMEM((_K, 16), _f32),
            pltpu.VMEM_SHARED((_NP, 16), _f32),
        ],
        compiler_params=_SC_PARAMS,
    )
    return kern(dstF)


def _prop_body(table, srcF, dstF, out, dstv, srcsh,
               b0, b1, b2, b3,
               acc, gsA, gsB, ssA, ssB, *, w, ch, slabs0, slabs1):
    cid = lax.axis_index("c")
    sid = lax.axis_index("s")
    bufs = (b0, b1, b2, b3)
    ng = ch // 4
    pltpu.sync_copy(srcF.at[sid], srcsh)
    pltpu.sync_copy(dstF.at[sid], dstv)

    def _fill(r, _):
        for c in range(w // 16):
            b0[r, pl.ds(c * 16, 16)] = jnp.zeros((16,), _f32)
        return 0

    def _gather(j, b, sem):
        pltpu.async_copy(table.at[srcsh.at[j]], bufs[b], sem)

    def _drain(sem, b):
        pltpu.make_async_copy(table.at[pl.ds(0, _K)], bufs[b], sem).wait()

    def _scatter(j, b, sem):
        del sem
        pltpu.sync_copy(bufs[b], acc.at[dstv.at[j]], add=True)

    def _group(g, _):
        j0 = 4 * g

        for b in range(2):
            _gather(j0 + 2 + b, 2 + b, gsB)
        for b in range(2):
            _drain(gsA, b)
        for b in range(2):
            _scatter(j0 + b, b, ssA)

        @pl.when(g + 1 < ng)
        def _():
            for b in range(2):
                _gather(j0 + 4 + b, b, gsA)
        for b in range(2):
            _drain(gsB, 2 + b)
        for b in range(2):
            _scatter(j0 + 2 + b, 2 + b, ssB)
        return 0

    def _one_slab(s, off):
        def _shift(j, _):
            for c in range(_K // 16):
                srcsh[j, pl.ds(c * 16, 16)] = \
                    srcsh[j, pl.ds(c * 16, 16)] + off
            return 0

        if off is not None:
            lax.fori_loop(0, ch, _shift, 0)
        lax.fori_loop(0, _K, _fill, 0)
        for z in range(_STRIPE // _K):
            pltpu.sync_copy(b0, acc.at[pl.ds(sid * _STRIPE + z * _K, _K)])
        for b in range(2):
            _gather(b, b, gsA)
        plsc.subcore_barrier()
        lax.fori_loop(0, ng, _group, 0)
        plsc.subcore_barrier()
        pltpu.sync_copy(acc.at[pl.ds(sid * _STRIPE, _STRIPE)],
                        out.at[s, pl.ds(sid * _STRIPE, _STRIPE)])

    num_slabs = len(slabs0) + len(slabs1)
    if slabs1 and len(slabs0) == len(slabs1):
        half = num_slabs // 2
        base = cid * half
        for i in range(half):
            s = base + i
            off = base * jnp.int32(_NP) if i == 0 else jnp.int32(_NP)
            _one_slab(s, off)
    else:
        for cv, my_slabs in ((0, slabs0), (1, slabs1)):
            if not my_slabs:
                continue

            @pl.when(cid == cv)
            def _(my_slabs=my_slabs):
                prev = 0
                for s in my_slabs:
                    d = s - prev
                    _one_slab(s, jnp.int32(d * _NP) if d else None)
                    prev = s


def _sc_prop(table_flat, srcF, dstF, num_slabs, w, ch, n1):
    slabs0 = tuple(range(num_slabs - n1))
    slabs1 = tuple(range(num_slabs - n1, num_slabs))
    mesh = plsc.VectorSubcoreMesh(core_axis_name="c", subcore_axis_name="s")
    kern = pl.kernel(
        functools.partial(_prop_body, w=w, ch=ch,
                          slabs0=slabs0, slabs1=slabs1),
        out_type=jax.ShapeDtypeStruct((num_slabs, _NP, w), _f32),
        mesh=mesh,
        scratch_types=[
            pltpu.VMEM((ch, _K), jnp.int32),
            pltpu.VMEM((ch, _K), jnp.int32),
        ] + [pltpu.VMEM((_K, w), _f32) for _ in range(4)] + [
            pltpu.VMEM_SHARED((_NP, w), _f32),
            pltpu.SemaphoreType.DMA,
            pltpu.SemaphoreType.DMA,
            pltpu.SemaphoreType.DMA,
            pltpu.SemaphoreType.DMA,
        ],
        compiler_params=_SC_PARAMS,
    )
    return kern(table_flat, srcF, dstF)


_TC_PARAMS = pltpu.CompilerParams(dimension_semantics=("arbitrary",))


def _u_body(degp, xpad, u_ref, xp_ref):
    deg = degp[0, :, 0:1] + degp[1, :, 0:1] + 1.0
    u = lax.rsqrt(deg)
    u_ref[...] = u
    xs = xpad[...] * u
    xp_ref[0] = xs[:, :_W]
    xp_ref[1] = xs[:, _W:]


def _tc_u(degp, xpad):
    return pl.pallas_call(
        _u_body,
        grid=(_MG,),
        in_specs=[
            pl.BlockSpec((_NC, _BM, 16), lambda i: (0, i, 0)),
            pl.BlockSpec((_BM, 128), lambda i: (i, 0)),
        ],
        out_specs=[
            pl.BlockSpec((_BM, 1), lambda i: (i, 0)),
            pl.BlockSpec((2, _BM, _W), lambda i: (0, i, 0)),
        ],
        out_shape=[
            jax.ShapeDtypeStruct((_NP, 1), _f32),
            jax.ShapeDtypeStruct((2, _NP, _W), _f32),
        ],
        compiler_params=_TC_PARAMS,
    )(degp, xpad)


def _l1_body(p1, xp, u, w1, b1, out):
    z = jnp.concatenate(
        [p1[k] + xp[k] for k in range(2)], axis=1) * u[...]
    h = jnp.dot(z, w1[...], preferred_element_type=_f32) + b1[...]
    out[...] = jnp.maximum(h, 0.0)


def _tc_l1(p1, xp, u, w1, b1):
    dout = w1.shape[1]
    return pl.pallas_call(
        _l1_body,
        grid=(_MG,),
        in_specs=[
            pl.BlockSpec((2, _BM, _W), lambda i: (0, i, 0)),
            pl.BlockSpec((2, _BM, _W), lambda i: (0, i, 0)),
            pl.BlockSpec((_BM, 1), lambda i: (i, 0)),
            pl.BlockSpec(w1.shape, lambda i: (0, 0)),
            pl.BlockSpec(b1.shape, lambda i: (0, 0)),
        ],
        out_specs=pl.BlockSpec((_BM, dout), lambda i: (i, 0)),
        out_shape=jax.ShapeDtypeStruct((_NP, dout), _f32),
        compiler_params=_TC_PARAMS,
    )(p1, xp, u, w1, b1)


def _l2_body(h1, u, w2, out, *, ns):
    g = jnp.dot(h1[...], w2[...], preferred_element_type=_f32) * u[...]
    for j in range(ns):
        out[j] = g[:, _W * j:_W * (j + 1)]


def _tc_l2(h1, u, w2):
    din = w2.shape[0]
    ns = w2.shape[1] // _W
    return pl.pallas_call(
        functools.partial(_l2_body, ns=ns),
        grid=(_MG,),
        in_specs=[
            pl.BlockSpec((_BM, din), lambda i: (i, 0)),
            pl.BlockSpec((_BM, 1), lambda i: (i, 0)),
            pl.BlockSpec(w2.shape, lambda i: (0, 0)),
        ],
        out_specs=pl.BlockSpec((ns, _BM, _W), lambda i: (0, i, 0)),
        out_shape=jax.ShapeDtypeStruct((ns, _NP, _W), _f32),
        compiler_params=_TC_PARAMS,
    )(h1, u, w2)


def _mid_body(pp, gg, u, b, w, out, *, ks, dout):
    uu = u[...]
    acc = None
    wm = w[...]
    for k in range(0, ks, 2):
        hk = jnp.concatenate(
            [jnp.maximum((pp[k + j] + gg[k + j]) * uu
                         + b[k + j], 0.0) for j in range(2)], axis=1)
        t = jnp.dot(hk, wm[_W * k:_W * (k + 2), :],
                    preferred_element_type=_f32)
        acc = t if acc is None else acc + t
    g = acc * uu
    if dout > _W:
        for j in range(dout // _W):
            out[j] = g[:, _W * j:_W * (j + 1)]
    elif dout == _W:
        out[0] = g[:, :_W // 2]
        out[1] = g[:, _W // 2:]
    else:
        out[...] = g


def _tc_mid(pp, gg, u, b, w):
    ks = gg.shape[0]
    dout = w.shape[1]
    if dout > _W:
        out_spec = pl.BlockSpec((dout // _W, _BM, _W), lambda i: (0, i, 0))
        out_shape = jax.ShapeDtypeStruct((dout // _W, _NP, _W), _f32)
    elif dout == _W:
        out_spec = pl.BlockSpec((2, _BM, _W // 2), lambda i: (0, i, 0))
        out_shape = jax.ShapeDtypeStruct((2, _NP, _W // 2), _f32)
    else:
        out_spec = pl.BlockSpec((_BM, dout), lambda i: (i, 0))
        out_shape = jax.ShapeDtypeStruct((_NP, dout), _f32)
    return pl.pallas_call(
        functools.partial(_mid_body, ks=ks, dout=dout),
        grid=(_MG,),
        in_specs=[
            pl.BlockSpec((ks, _BM, _W), lambda i: (0, i, 0)),
            pl.BlockSpec((ks, _BM, _W), lambda i: (0, i, 0)),
            pl.BlockSpec((_BM, 1), lambda i: (i, 0)),
            pl.BlockSpec(b.shape, lambda i: (0, 0)),
            pl.BlockSpec(w.shape, lambda i: (0, 0)),
        ],
        out_specs=out_spec,
        out_shape=out_shape,
        compiler_params=_TC_PARAMS,
    )(pp, gg, u, b, w)


def _nar_body(pp, gg, u, b, w, out):
    uu = u[...]
    hk = jnp.concatenate([pp[k] + gg[k] for k in range(2)], axis=1)
    hk = jnp.maximum(hk * uu + b[...], 0.0)
    out[...] = jnp.dot(hk, w[...], preferred_element_type=_f32) * uu


def _tc_narrow(pp, gg, u, b, w):
    hw = gg.shape[2]
    dout = w.shape[1]
    return pl.pallas_call(
        _nar_body,
        grid=(_MG,),
        in_specs=[
            pl.BlockSpec((2, _BM, hw), lambda i: (0, i, 0)),
            pl.BlockSpec((2, _BM, hw), lambda i: (0, i, 0)),
            pl.BlockSpec((_BM, 1), lambda i: (i, 0)),
            pl.BlockSpec(b.shape, lambda i: (0, 0)),
            pl.BlockSpec(w.shape, lambda i: (0, 0)),
        ],
        out_specs=pl.BlockSpec((_BM, dout), lambda i: (i, 0)),
        out_shape=jax.ShapeDtypeStruct((_NP, dout), _f32),
        compiler_params=_TC_PARAMS,
    )(pp, gg, u, b, w)


def _fin_body(pp, gg, u, b5, wl, bl, out, acc):
    m = pl.program_id(0)
    h5 = (pp[...] + gg[...]) * u[...] + b5[...]
    rows = m * _BM + lax.broadcasted_iota(jnp.int32, (_BM, 1), 0)
    h5 = jnp.where(rows < _N, h5, 0.0)
    part = jnp.sum(h5, axis=0, keepdims=True)

    @pl.when(m == 0)
    def _():
        acc[...] = jnp.zeros_like(acc)

    acc[...] += part

    @pl.when(m == _MG - 1)
    def _():
        pooled = acc[...] * (1.0 / _N)
        out[...] = jnp.dot(pooled, wl[...], preferred_element_type=_f32) \
            + bl[...]


def _tc_final(pp, gg, u, b5, wl, bl):
    din = gg.shape[1]
    return pl.pallas_call(
        _fin_body,
        grid=(_MG,),
        in_specs=[
            pl.BlockSpec((_BM, din), lambda i: (i, 0)),
            pl.BlockSpec((_BM, din), lambda i: (i, 0)),
            pl.BlockSpec((_BM, 1), lambda i: (i, 0)),
            pl.BlockSpec(b5.shape, lambda i: (0, 0)),
            pl.BlockSpec(wl.shape, lambda i: (0, 0)),
            pl.BlockSpec(bl.shape, lambda i: (0, 0)),
        ],
        out_specs=pl.BlockSpec((1, 3), lambda i: (0, 0)),
        out_shape=jax.ShapeDtypeStruct((1, 3), _f32),
        scratch_shapes=[pltpu.VMEM((1, din), _f32)],
        compiler_params=_TC_PARAMS,
    )(pp, gg, u, b5, wl, bl)


def kernel(x, edge_index, W1, b1, W2, b2, W3, b3, W4, b4, W5, b5, Wl, bl):
    e = edge_index.shape[1]
    ch = -(-e // (_K * _NS))
    ch = -(-ch // 4) * 4
    tot = _NS * ch
    pad = tot * _K - e

    src = jnp.concatenate([edge_index[0], jnp.zeros((pad,), jnp.int32)])
    dst = jnp.concatenate([edge_index[1], jnp.full((pad,), _N, jnp.int32)])
    srcF = src.reshape(_NS, ch, _K)
    dstF = dst.reshape(_NS, ch, _K)

    xpad = jnp.pad(x, ((0, _NP - _N), (0, 0)))

    degp = _sc_degree(dstF, ch)
    u, xp = _tc_u(degp, xpad)

    p1 = _sc_prop(xp.reshape(-1, _W), srcF, dstF, 2, _W, ch, 1)
    h1 = _tc_l1(p1, xp, u, W1, b1.reshape(1, -1))

    g2 = _tc_l2(h1, u, W2)
    p2 = _sc_prop(g2.reshape(-1, _W), srcF, dstF, 16, _W, ch, 8)

    g3 = _tc_mid(p2, g2, u, b2.reshape(-1, _W), W3)
    p3 = _sc_prop(g3.reshape(-1, _W), srcF, dstF, 4, _W, ch, 2)

    g4 = _tc_mid(p3, g3, u, b3.reshape(-1, _W), W4)
    p4 = _sc_prop(g4.reshape(-1, 32), srcF, dstF, 2, 32, ch, 1)

    g5 = _tc_narrow(p4, g4, u, b4.reshape(1, -1), W5)
    p5 = _sc_prop(g5, srcF, dstF, 1, 32, ch, 0)[0]

    return _tc_final(p5, g5, u, b5.reshape(1, -1), Wl, bl.reshape(1, -1))
